# Initial kernel scaffold; baseline (speedup 1.0000x reference)
import jax
import jax.numpy as jnp
from jax.experimental import pallas as pl


def kernel(edge_attr_sca, edge_attr_vec, edge_index, pos_compose, index_real_cps_edge_for_atten_i, index_real_cps_edge_for_atten_j, tri_edge_index, tri_edge_feat, params):
    raise NotImplementedError("write your pallas kernel here")


# speedup vs baseline: 3.0500x; 3.0500x over previous
import jax
import jax.numpy as jnp
from jax.experimental import pallas as pl

HID_SCA, HID_VEC = 128, 32
KEY_SCA, KEY_VEC = 64, 16
NUM_HEADS = 16
NUM_EDGE_TYPES = 3
CUTOFF = 10.0

BN = 2560
BA = 2560


def _qkv_kernel(sca_ref, vec_ref,
                qWvT, qWv2T, qWsvT, qWssT, qGwT, qGb,
                kWvT, kWv2T, kWsvT, kWssT, kGwT, kGb,
                vWvT, vWv2T, vWsvT, vWssT, vGwT, vGb,
                qs_o, qv_o, ks_o, kv_o, vs_o, vv_o):
    sca = sca_ref[...]
    vx = vec_ref[:, 0:32]
    vy = vec_ref[:, 32:64]
    vz = vec_ref[:, 64:96]

    def gv(WvT, Wv2T, WsvT, WssT, GwT, Gb):
        ix = vx @ WvT[...]
        iy = vy @ WvT[...]
        iz = vz @ WvT[...]
        vnorm = jnp.sqrt(ix * ix + iy * iy + iz * iz + 1e-12)
        out_s = vnorm @ WsvT[...] + sca @ WssT[...]
        ox = ix @ Wv2T[...]
        oy = iy @ Wv2T[...]
        oz = iz @ Wv2T[...]
        gate = jax.nn.sigmoid(out_s @ GwT[...] + Gb[...])
        return out_s, gate * ox, gate * oy, gate * oz

    qs, qx, qy, qz = gv(qWvT, qWv2T, qWsvT, qWssT, qGwT, qGb)
    ks, kx, ky, kz = gv(kWvT, kWv2T, kWsvT, kWssT, kGwT, kGb)
    vs, vvx, vvy, vvz = gv(vWvT, vWv2T, vWsvT, vWssT, vGwT, vGb)
    qs_o[...] = qs
    qv_o[...] = jnp.concatenate([qx, qy, qz], axis=1)
    ks_o[...] = ks
    kv_o[...] = jnp.concatenate([kx, ky, kz], axis=1)
    vs_o[...] = vs
    vv_o[...] = jnp.concatenate([vvx, vvy, vvz], axis=1)


def _logits_kernel(vec_ref, trif_ref, qis_ref, kjs_ref, qiv_ref, kjv_ref,
                   offs_ref, coeff_ref, u2_ref, g2_ref,
                   WsvT, WssT, GwT, Gb,
                   ls_o, lv_o):
    vx = vec_ref[:, 0:1]
    vy = vec_ref[:, 1:2]
    vz = vec_ref[:, 2:3]
    d2 = vx * vx + vy * vy + vz * vz
    dist = jnp.sqrt(d2 + 1e-12)
    inv = 1.0 / (dist + 1e-7)
    s2 = d2 * inv * inv

    coeff = coeff_ref[...]
    diff = dist - offs_ref[...]
    gauss = jnp.exp(coeff * diff * diff)

    r3 = jax.lax.broadcasted_iota(jnp.int32, (8, 128), 0)
    c3 = jax.lax.broadcasted_iota(jnp.int32, (8, 128), 1)
    P = jnp.where(c3 == r3 + 125, 1.0, 0.0).astype(jnp.float32)
    trif = trif_ref[...]
    tri128 = trif @ P
    col = jax.lax.broadcasted_iota(jnp.int32, (1, 128), 1)
    sca_feat = jnp.where(col < 125, gauss, tri128)

    vec_norm = jnp.sqrt(s2 * u2_ref[...] + 1e-12)
    out_sca = vec_norm @ WsvT[...] + sca_feat @ WssT[...]
    gate = jax.nn.sigmoid(out_sca @ GwT[...] + Gb[...])
    bias_v = gate * gate * g2_ref[...] * s2

    rG = jax.lax.broadcasted_iota(jnp.int32, (64, 16), 0)
    cG = jax.lax.broadcasted_iota(jnp.int32, (64, 16), 1)
    G64 = jnp.where(rG // 4 == cG, 1.0, 0.0).astype(jnp.float32)
    qk_s = (qis_ref[...] * kjs_ref[...]) @ G64
    pv = qiv_ref[...] * kjv_ref[...]
    qk_v = pv[:, 0:16] + pv[:, 16:32] + pv[:, 32:48]

    ls_o[...] = out_sca + qk_s
    lv_o[...] = bias_v + qk_v


def _weight_kernel(ls_ref, ms_ref, ds_ref, lv_ref, mv_ref, dv_ref,
                   vjs_ref, vjv_ref, ws_o, wv_o):
    a_s = jnp.exp(ls_ref[...] - ms_ref[...]) / (ds_ref[...] + 1e-16)
    a_v = jnp.exp(lv_ref[...] - mv_ref[...]) / (dv_ref[...] + 1e-16)
    r8 = jax.lax.broadcasted_iota(jnp.int32, (16, 128), 0)
    c8 = jax.lax.broadcasted_iota(jnp.int32, (16, 128), 1)
    E8 = jnp.where(c8 // 8 == r8, 1.0, 0.0).astype(jnp.float32)
    r2 = jax.lax.broadcasted_iota(jnp.int32, (16, 32), 0)
    c2 = jax.lax.broadcasted_iota(jnp.int32, (16, 32), 1)
    E2 = jnp.where(c2 // 2 == r2, 1.0, 0.0).astype(jnp.float32)
    a128 = a_s @ E8
    a32 = a_v @ E2
    ws_o[...] = a128 * vjs_ref[...]
    vjv = vjv_ref[...]
    wv_o[...] = jnp.concatenate(
        [a32 * vjv[:, 0:32], a32 * vjv[:, 32:64], a32 * vjv[:, 64:96]], axis=1)


def _final_kernel(sca_ref, vec_ref, aggs_ref, aggv_ref,
                  lsw, lsb, lvw, lvb, os_o, ov_o):
    s = sca_ref[...] + aggs_ref[...]
    mu = jnp.mean(s, axis=-1, keepdims=True)
    var = jnp.mean((s - mu) ** 2, axis=-1, keepdims=True)
    os_o[...] = (s - mu) / jnp.sqrt(var + 1e-5) * lsw[...] + lsb[...]
    v = vec_ref[...] + aggv_ref[...]
    muv = jnp.mean(v, axis=-1, keepdims=True)
    varv = jnp.mean((v - muv) ** 2, axis=-1, keepdims=True)
    ov_o[...] = (v - muv) / jnp.sqrt(varv + 1e-5) * lvw[...] + lvb[...]


def _full(a):
    return pl.BlockSpec(a.shape, lambda i: (0,) * a.ndim)


def kernel(edge_attr_sca, edge_attr_vec, edge_index, pos_compose,
           index_real_cps_edge_for_atten_i, index_real_cps_edge_for_atten_j,
           tri_edge_index, tri_edge_feat, params):
    N = edge_attr_sca.shape[0]
    A = index_real_cps_edge_for_atten_i.shape[0]
    idx_i = index_real_cps_edge_for_atten_i
    idx_j = index_real_cps_edge_for_atten_j

    vec96 = edge_attr_vec.transpose(0, 2, 1).reshape(N, 96)

    def prep(p, out_s):
        Ws = p['lin_scalar']
        return [p['lin_vector'].T, p['lin_vector2'].T,
                Ws[:, :32].T, Ws[:, 32:].T,
                p['gate_w'].T, p['gate_b'][None, :]]

    pq = prep(params['q'], KEY_SCA)
    pk = prep(params['k'], KEY_SCA)
    pv = prep(params['v'], HID_SCA)
    nb = N // BN
    f32 = jnp.float32
    qs, qv, ks, kv, vs, vv = pl.pallas_call(
        _qkv_kernel,
        grid=(nb,),
        in_specs=[pl.BlockSpec((BN, 128), lambda i: (i, 0)),
                  pl.BlockSpec((BN, 96), lambda i: (i, 0))]
                 + [_full(a) for a in pq + pk + pv],
        out_specs=[pl.BlockSpec((BN, 64), lambda i: (i, 0)),
                   pl.BlockSpec((BN, 48), lambda i: (i, 0)),
                   pl.BlockSpec((BN, 64), lambda i: (i, 0)),
                   pl.BlockSpec((BN, 48), lambda i: (i, 0)),
                   pl.BlockSpec((BN, 128), lambda i: (i, 0)),
                   pl.BlockSpec((BN, 96), lambda i: (i, 0))],
        out_shape=[jax.ShapeDtypeStruct((N, 64), f32),
                   jax.ShapeDtypeStruct((N, 48), f32),
                   jax.ShapeDtypeStruct((N, 64), f32),
                   jax.ShapeDtypeStruct((N, 48), f32),
                   jax.ShapeDtypeStruct((N, 128), f32),
                   jax.ShapeDtypeStruct((N, 96), f32)],
    )(edge_attr_sca, vec96, *pq, *pk, *pv)

    pa = pos_compose[tri_edge_index[0]]
    pb = pos_compose[tri_edge_index[1]]
    vector = pa - pb
    trif8 = jnp.pad(tri_edge_feat, ((0, 0), (0, 5)))

    num_g = HID_SCA - NUM_EDGE_TYPES
    offset = jnp.linspace(0.0, CUTOFF, num_g)
    coeff = (-0.5 / (offset[1] - offset[0]) ** 2).reshape(1, 1)
    offs128 = jnp.concatenate([offset, jnp.zeros((3,), f32)]).reshape(1, 128)

    pb_ = params['gv_bias']
    u = pb_['lin_vector'] @ params['vec_exp_w']
    g = pb_['lin_vector2'] @ u
    u2 = (u * u)[None, :]
    g2 = (g * g)[None, :]
    Wsb = pb_['lin_scalar']
    bias_params = [offs128, coeff, u2, g2, Wsb[:, :32].T, Wsb[:, 32:].T,
                   pb_['gate_w'].T, pb_['gate_b'][None, :]]

    qis = qs[idx_i]
    qiv = qv[idx_i]
    kjs = ks[idx_j]
    kjv = kv[idx_j]

    na = A // BA
    ls, lv = pl.pallas_call(
        _logits_kernel,
        grid=(na,),
        in_specs=[pl.BlockSpec((BA, 3), lambda i: (i, 0)),
                  pl.BlockSpec((BA, 8), lambda i: (i, 0)),
                  pl.BlockSpec((BA, 64), lambda i: (i, 0)),
                  pl.BlockSpec((BA, 64), lambda i: (i, 0)),
                  pl.BlockSpec((BA, 48), lambda i: (i, 0)),
                  pl.BlockSpec((BA, 48), lambda i: (i, 0))]
                 + [_full(a) for a in bias_params],
        out_specs=[pl.BlockSpec((BA, 16), lambda i: (i, 0)),
                   pl.BlockSpec((BA, 16), lambda i: (i, 0))],
        out_shape=[jax.ShapeDtypeStruct((A, 16), f32),
                   jax.ShapeDtypeStruct((A, 16), f32)],
    )(vector, trif8, qis, kjs, qiv, kjv, *bias_params)

    m_s = jax.ops.segment_max(ls, idx_i, num_segments=N, indices_are_sorted=True)
    m_s = jnp.where(jnp.isfinite(m_s), m_s, 0.0)
    m_v = jax.ops.segment_max(lv, idx_i, num_segments=N, indices_are_sorted=True)
    m_v = jnp.where(jnp.isfinite(m_v), m_v, 0.0)
    ex_s = jnp.exp(ls - m_s[idx_i])
    ex_v = jnp.exp(lv - m_v[idx_i])
    d_s = jax.ops.segment_sum(ex_s, idx_i, num_segments=N, indices_are_sorted=True)
    d_v = jax.ops.segment_sum(ex_v, idx_i, num_segments=N, indices_are_sorted=True)

    vjs = vs[idx_j]
    vjv = vv[idx_j]
    ws, wv = pl.pallas_call(
        _weight_kernel,
        grid=(na,),
        in_specs=[pl.BlockSpec((BA, 16), lambda i: (i, 0))] * 6
                 + [pl.BlockSpec((BA, 128), lambda i: (i, 0)),
                    pl.BlockSpec((BA, 96), lambda i: (i, 0))],
        out_specs=[pl.BlockSpec((BA, 128), lambda i: (i, 0)),
                   pl.BlockSpec((BA, 96), lambda i: (i, 0))],
        out_shape=[jax.ShapeDtypeStruct((A, 128), f32),
                   jax.ShapeDtypeStruct((A, 96), f32)],
    )(ls, m_s[idx_i], d_s[idx_i], lv, m_v[idx_i], d_v[idx_i], vjs, vjv)

    agg_s = jax.ops.segment_sum(ws, idx_i, num_segments=N, indices_are_sorted=True)
    agg_v = jax.ops.segment_sum(wv, idx_i, num_segments=N, indices_are_sorted=True)

    ln = [params['ln_sca_w'][None, :], params['ln_sca_b'][None, :],
          params['ln_vec_w'].T.reshape(1, 96), params['ln_vec_b'].T.reshape(1, 96)]
    out_s, out_v96 = pl.pallas_call(
        _final_kernel,
        grid=(nb,),
        in_specs=[pl.BlockSpec((BN, 128), lambda i: (i, 0)),
                  pl.BlockSpec((BN, 96), lambda i: (i, 0)),
                  pl.BlockSpec((BN, 128), lambda i: (i, 0)),
                  pl.BlockSpec((BN, 96), lambda i: (i, 0))]
                 + [_full(a) for a in ln],
        out_specs=[pl.BlockSpec((BN, 128), lambda i: (i, 0)),
                   pl.BlockSpec((BN, 96), lambda i: (i, 0))],
        out_shape=[jax.ShapeDtypeStruct((N, 128), f32),
                   jax.ShapeDtypeStruct((N, 96), f32)],
    )(edge_attr_sca, vec96, agg_s, agg_v, *ln)

    out_v = out_v96.reshape(N, 3, 32).transpose(0, 2, 1)
    return out_s, out_v


# speedup vs baseline: 20.4046x; 6.6899x over previous
import jax
import jax.numpy as jnp
from jax.experimental import pallas as pl

HID_SCA, HID_VEC = 128, 32
KEY_SCA, KEY_VEC = 64, 16
NUM_HEADS = 16
NUM_EDGE_TYPES = 3
CUTOFF = 10.0

BN = 2560
BA = 2560


def _head_expand(n):
    r = jax.lax.broadcasted_iota(jnp.int32, (16, n), 0)
    c = jax.lax.broadcasted_iota(jnp.int32, (16, n), 1)
    return jnp.where(c // (n // 16) == r, 1.0, 0.0).astype(jnp.float32)


def _qkv_kernel(sca_ref, vec_ref,
                qWvT, qWv2T, qWsvT, qWssT, qGwT, qGb,
                kWvT, kWv2T, kWsvT, kWssT, kGwT, kGb,
                vWvT, vWv2T, vWsvT, vWssT, vGwT, vGb,
                q_o, k_o, v_o):
    sca = sca_ref[...]
    vx = vec_ref[:, 0:32]
    vy = vec_ref[:, 32:64]
    vz = vec_ref[:, 64:96]

    def gv(WvT, Wv2T, WsvT, WssT, GwT, Gb):
        ix = vx @ WvT[...]
        iy = vy @ WvT[...]
        iz = vz @ WvT[...]
        vnorm = jnp.sqrt(ix * ix + iy * iy + iz * iz + 1e-12)
        out_s = vnorm @ WsvT[...] + sca @ WssT[...]
        ox = ix @ Wv2T[...]
        oy = iy @ Wv2T[...]
        oz = iz @ Wv2T[...]
        gate = jax.nn.sigmoid(out_s @ GwT[...] + Gb[...])
        return jnp.concatenate([out_s, gate * ox, gate * oy, gate * oz], axis=1)

    q_o[...] = gv(qWvT, qWv2T, qWsvT, qWssT, qGwT, qGb)
    k_o[...] = gv(kWvT, kWv2T, kWsvT, kWssT, kGwT, kGb)
    v_o[...] = gv(vWvT, vWv2T, vWsvT, vWssT, vGwT, vGb)


def _logits_kernel(vec_ref, trif_ref, qi_ref, kj_ref,
                   offs_ref, coeff_ref, u2_ref, g2_ref,
                   WsvT, WssT, GwT, Gb,
                   l_o):
    vx = vec_ref[:, 0:1]
    vy = vec_ref[:, 1:2]
    vz = vec_ref[:, 2:3]
    d2 = vx * vx + vy * vy + vz * vz
    dist = jnp.sqrt(d2 + 1e-12)
    inv = 1.0 / (dist + 1e-7)
    s2 = d2 * inv * inv

    coeff = coeff_ref[...]
    diff = dist - offs_ref[...]
    gauss = jnp.exp(coeff * diff * diff)

    r3 = jax.lax.broadcasted_iota(jnp.int32, (8, 128), 0)
    c3 = jax.lax.broadcasted_iota(jnp.int32, (8, 128), 1)
    P = jnp.where(c3 == r3 + 125, 1.0, 0.0).astype(jnp.float32)
    tri128 = trif_ref[...] @ P
    col = jax.lax.broadcasted_iota(jnp.int32, (1, 128), 1)
    sca_feat = jnp.where(col < 125, gauss, tri128)

    vec_norm = jnp.sqrt(s2 * u2_ref[...] + 1e-12)
    out_sca = vec_norm @ WsvT[...] + sca_feat @ WssT[...]
    gate = jax.nn.sigmoid(out_sca @ GwT[...] + Gb[...])
    bias_v = gate * gate * g2_ref[...] * s2

    rG = jax.lax.broadcasted_iota(jnp.int32, (64, 16), 0)
    cG = jax.lax.broadcasted_iota(jnp.int32, (64, 16), 1)
    G64 = jnp.where(rG // 4 == cG, 1.0, 0.0).astype(jnp.float32)
    qi = qi_ref[...]
    kj = kj_ref[...]
    qk_s = (qi[:, 0:64] * kj[:, 0:64]) @ G64
    pv = qi[:, 64:112] * kj[:, 64:112]
    qk_v = pv[:, 0:16] + pv[:, 16:32] + pv[:, 32:48]

    l_o[...] = jnp.concatenate([out_sca + qk_s, bias_v + qk_v], axis=1)


def _weight_kernel(l_ref, mg_ref, vj_ref, w_o):
    ex = jnp.exp(l_ref[...] - mg_ref[...])
    a128 = ex[:, 0:16] @ _head_expand(128)
    a32 = ex[:, 16:32] @ _head_expand(32)
    vj = vj_ref[...]
    w_o[...] = jnp.concatenate(
        [ex, a128 * vj[:, 0:128],
         a32 * vj[:, 128:160], a32 * vj[:, 160:192], a32 * vj[:, 192:224]],
        axis=1)


def _final_kernel(sca_ref, vec_ref, agg_ref, lsw, lsb, lvw, lvb, os_o, ov_o):
    agg = agg_ref[...]
    den_s = agg[:, 0:16] @ _head_expand(128) + 1e-16
    den_v = agg[:, 16:32] @ _head_expand(32) + 1e-16
    s = sca_ref[...] + agg[:, 32:160] / den_s
    mu = jnp.mean(s, axis=-1, keepdims=True)
    var = jnp.mean((s - mu) ** 2, axis=-1, keepdims=True)
    os_o[...] = (s - mu) / jnp.sqrt(var + 1e-5) * lsw[...] + lsb[...]
    v = vec_ref[...] + jnp.concatenate(
        [agg[:, 160:192] / den_v, agg[:, 192:224] / den_v,
         agg[:, 224:256] / den_v], axis=1)
    muv = jnp.mean(v, axis=-1, keepdims=True)
    varv = jnp.mean((v - muv) ** 2, axis=-1, keepdims=True)
    ov_o[...] = (v - muv) / jnp.sqrt(varv + 1e-5) * lvw[...] + lvb[...]


def _full(a):
    return pl.BlockSpec(a.shape, lambda i: (0,) * a.ndim)


def kernel(edge_attr_sca, edge_attr_vec, edge_index, pos_compose,
           index_real_cps_edge_for_atten_i, index_real_cps_edge_for_atten_j,
           tri_edge_index, tri_edge_feat, params):
    N = edge_attr_sca.shape[0]
    A = index_real_cps_edge_for_atten_i.shape[0]
    idx_i = index_real_cps_edge_for_atten_i
    idx_j = index_real_cps_edge_for_atten_j

    vec96 = edge_attr_vec.transpose(0, 2, 1).reshape(N, 96)

    def prep(p):
        Ws = p['lin_scalar']
        return [p['lin_vector'].T, p['lin_vector2'].T,
                Ws[:, :32].T, Ws[:, 32:].T,
                p['gate_w'].T, p['gate_b'][None, :]]

    pq = prep(params['q'])
    pk = prep(params['k'])
    pv = prep(params['v'])
    nb = N // BN
    f32 = jnp.float32
    q_cat, k_cat, v_cat = pl.pallas_call(
        _qkv_kernel,
        grid=(nb,),
        in_specs=[pl.BlockSpec((BN, 128), lambda i: (i, 0)),
                  pl.BlockSpec((BN, 96), lambda i: (i, 0))]
                 + [_full(a) for a in pq + pk + pv],
        out_specs=[pl.BlockSpec((BN, 112), lambda i: (i, 0)),
                   pl.BlockSpec((BN, 112), lambda i: (i, 0)),
                   pl.BlockSpec((BN, 224), lambda i: (i, 0))],
        out_shape=[jax.ShapeDtypeStruct((N, 112), f32),
                   jax.ShapeDtypeStruct((N, 112), f32),
                   jax.ShapeDtypeStruct((N, 224), f32)],
    )(edge_attr_sca, vec96, *pq, *pk, *pv)

    vector = pos_compose[tri_edge_index[0]] - pos_compose[tri_edge_index[1]]
    trif8 = jnp.pad(tri_edge_feat, ((0, 0), (0, 5)))

    num_g = HID_SCA - NUM_EDGE_TYPES
    offset = jnp.linspace(0.0, CUTOFF, num_g)
    coeff = (-0.5 / (offset[1] - offset[0]) ** 2).reshape(1, 1)
    offs128 = jnp.concatenate([offset, jnp.zeros((3,), f32)]).reshape(1, 128)

    pb_ = params['gv_bias']
    u = pb_['lin_vector'] @ params['vec_exp_w']
    g = pb_['lin_vector2'] @ u
    Wsb = pb_['lin_scalar']
    bias_params = [offs128, coeff, (u * u)[None, :], (g * g)[None, :],
                   Wsb[:, :32].T, Wsb[:, 32:].T,
                   pb_['gate_w'].T, pb_['gate_b'][None, :]]

    qi = q_cat[idx_i]
    kj = k_cat[idx_j]
    vj = v_cat[idx_j]

    na = A // BA
    l_cat = pl.pallas_call(
        _logits_kernel,
        grid=(na,),
        in_specs=[pl.BlockSpec((BA, 3), lambda i: (i, 0)),
                  pl.BlockSpec((BA, 8), lambda i: (i, 0)),
                  pl.BlockSpec((BA, 112), lambda i: (i, 0)),
                  pl.BlockSpec((BA, 112), lambda i: (i, 0))]
                 + [_full(a) for a in bias_params],
        out_specs=pl.BlockSpec((BA, 32), lambda i: (i, 0)),
        out_shape=jax.ShapeDtypeStruct((A, 32), f32),
    )(vector, trif8, qi, kj, *bias_params)

    m = jax.ops.segment_max(l_cat, idx_i, num_segments=N, indices_are_sorted=True)
    m = jnp.where(jnp.isfinite(m), m, 0.0)

    w_cat = pl.pallas_call(
        _weight_kernel,
        grid=(na,),
        in_specs=[pl.BlockSpec((BA, 32), lambda i: (i, 0)),
                  pl.BlockSpec((BA, 32), lambda i: (i, 0)),
                  pl.BlockSpec((BA, 224), lambda i: (i, 0))],
        out_specs=pl.BlockSpec((BA, 256), lambda i: (i, 0)),
        out_shape=jax.ShapeDtypeStruct((A, 256), f32),
    )(l_cat, m[idx_i], vj)

    agg = jax.ops.segment_sum(w_cat, idx_i, num_segments=N, indices_are_sorted=True)

    ln = [params['ln_sca_w'][None, :], params['ln_sca_b'][None, :],
          params['ln_vec_w'].T.reshape(1, 96), params['ln_vec_b'].T.reshape(1, 96)]
    out_s, out_v96 = pl.pallas_call(
        _final_kernel,
        grid=(nb,),
        in_specs=[pl.BlockSpec((BN, 128), lambda i: (i, 0)),
                  pl.BlockSpec((BN, 96), lambda i: (i, 0)),
                  pl.BlockSpec((BN, 256), lambda i: (i, 0))]
                 + [_full(a) for a in ln],
        out_specs=[pl.BlockSpec((BN, 128), lambda i: (i, 0)),
                   pl.BlockSpec((BN, 96), lambda i: (i, 0))],
        out_shape=[jax.ShapeDtypeStruct((N, 128), f32),
                   jax.ShapeDtypeStruct((N, 96), f32)],
    )(edge_attr_sca, vec96, agg, *ln)

    out_v = out_v96.reshape(N, 3, 32).transpose(0, 2, 1)
    return out_s, out_v
